# baseline (speedup 1.0000x reference)
import functools

import jax
import jax.numpy as jnp
from jax.experimental import pallas as pl
from jax.experimental.pallas import tpu as pltpu

EMBED_DIM = 256
NUM_CODES = 1024
NUM_SUB = 16
DS = EMBED_DIM // NUM_SUB


def _vq_body(z_ref, cb_ref, zq_ref, idx_ref, loss_ref):
    n_sub = cb_ref.shape[0]
    m = cb_ref.shape[1]
    hw = z_ref.shape[3]

    iota_m = jax.lax.broadcasted_iota(jnp.int32, (m, hw), 0)
    part = jnp.zeros((1, 1), jnp.float32)
    idx_rows = []
    for n in range(n_sub):
        cb = cb_ref[n]
        zb = z_ref[0, n]
        c2 = jnp.sum(cb * cb, axis=1, keepdims=True)
        cross = jax.lax.dot_general(cb, zb, (((1,), (0,)), ((), ())))
        score = c2 - 2.0 * cross
        idx = jnp.argmin(score, axis=0).reshape(1, hw).astype(jnp.int32)
        idx_rows.append(idx)
        onehot = (iota_m == idx).astype(jnp.float32)
        zq = jax.lax.dot_general(cb, onehot, (((0,), (0,)), ((), ())))
        zq_ref[0, n] = zq
        r = zq - zb
        part = part + jnp.sum(r * r).reshape(1, 1)
    idx_mat = jnp.concatenate(idx_rows, axis=0)
    idx_ref[:, :] = idx_mat.T
    loss_ref[0, 0] = part


@functools.partial(jax.jit, static_argnames=())
def kernel(z, codebooks):
    B, D, H, W = z.shape
    N, M, ds = codebooks.shape
    HW = H * W
    T = B * HW
    z4 = z.reshape(B, N, ds, HW)

    zq4, idx, loss_acc = pl.pallas_call(
        _vq_body,
        grid=(B,),
        in_specs=[
            pl.BlockSpec((1, N, ds, HW), lambda b: (b, 0, 0, 0)),
            pl.BlockSpec((N, M, ds), lambda b: (0, 0, 0)),
        ],
        out_specs=[
            pl.BlockSpec((1, N, ds, HW), lambda b: (b, 0, 0, 0)),
            pl.BlockSpec((HW, N), lambda b: (b, 0)),
            pl.BlockSpec((1, 1, 1, 1), lambda b: (b, 0, 0, 0)),
        ],
        out_shape=[
            jax.ShapeDtypeStruct((B, N, ds, HW), jnp.float32),
            jax.ShapeDtypeStruct((T, N), jnp.int32),
            jax.ShapeDtypeStruct((B, 1, 1, 1), jnp.float32),
        ],
        compiler_params=pltpu.CompilerParams(
            dimension_semantics=("parallel",),
        ),
    )(z4, codebooks)

    z_q_out = zq4.reshape(B, D, H, W)
    loss = jnp.sum(loss_acc) / jnp.float32(N * T * ds)
    return (z_q_out, loss, loss, idx)


# speedup vs baseline: 1.0414x; 1.0414x over previous
import functools

import jax
import jax.numpy as jnp
from jax.experimental import pallas as pl
from jax.experimental.pallas import tpu as pltpu

EMBED_DIM = 256
NUM_CODES = 1024
NUM_SUB = 16
DS = EMBED_DIM // NUM_SUB


def _vq_body(z_ref, cb_ref, zq_ref, idx_ref, loss_ref):
    n_sub = cb_ref.shape[0]
    m = cb_ref.shape[1]
    ds = cb_ref.shape[2]
    h, w = z_ref.shape[2], z_ref.shape[3]
    hw = h * w

    iota_m = jax.lax.broadcasted_iota(jnp.int32, (m, hw), 0)
    part = jnp.zeros((1, 1), jnp.float32)
    idx_rows = []
    for n in range(n_sub):
        cb = cb_ref[n]
        zb = z_ref[0, n * ds:(n + 1) * ds].reshape(ds, hw)
        c2 = jnp.sum(cb * cb, axis=1, keepdims=True)
        cross = jax.lax.dot_general(cb, zb, (((1,), (0,)), ((), ())))
        score = c2 - 2.0 * cross
        idx = jnp.argmin(score, axis=0).reshape(1, hw).astype(jnp.int32)
        idx_rows.append(idx)
        onehot = (iota_m == idx).astype(jnp.float32)
        zq = jax.lax.dot_general(cb, onehot, (((0,), (0,)), ((), ())))
        zq_ref[0, n * ds:(n + 1) * ds] = zq.reshape(ds, h, w)
        r = zq - zb
        part = part + jnp.sum(r * r).reshape(1, 1)
    idx_mat = jnp.concatenate(idx_rows, axis=0)
    idx_ref[:, :] = idx_mat.T
    loss_ref[0, 0] = part


@functools.partial(jax.jit, static_argnames=())
def kernel(z, codebooks):
    B, D, H, W = z.shape
    N, M, ds = codebooks.shape
    HW = H * W
    T = B * HW

    zq_out, idx, loss_acc = pl.pallas_call(
        _vq_body,
        grid=(B,),
        in_specs=[
            pl.BlockSpec((1, D, H, W), lambda b: (b, 0, 0, 0)),
            pl.BlockSpec((N, M, ds), lambda b: (0, 0, 0)),
        ],
        out_specs=[
            pl.BlockSpec((1, D, H, W), lambda b: (b, 0, 0, 0)),
            pl.BlockSpec((HW, N), lambda b: (b, 0)),
            pl.BlockSpec((1, 1, 1, 1), lambda b: (b, 0, 0, 0)),
        ],
        out_shape=[
            jax.ShapeDtypeStruct((B, D, H, W), jnp.float32),
            jax.ShapeDtypeStruct((T, N), jnp.int32),
            jax.ShapeDtypeStruct((B, 1, 1, 1), jnp.float32),
        ],
        compiler_params=pltpu.CompilerParams(
            dimension_semantics=("parallel",),
        ),
    )(z, codebooks)

    z_q_out = zq_out
    loss = jnp.sum(loss_acc) / jnp.float32(N * T * ds)
    return (z_q_out, loss, loss, idx)
